# baseline (speedup 1.0000x reference)
import jax
import jax.numpy as jnp
from jax.experimental import pallas as pl
from jax.experimental.pallas import tpu as pltpu

N = 10000
D = 128
C = 16
BN = 1000
GRID = N // BN

_SELU_ALPHA = 1.6732632423543772848170429916717
_SELU_SCALE = 1.0507009873554804934193349852946


def _dmon_kernel(x_ref, w_ref, b_ref, pooled_ref, assign_ref, s_ref):
    i = pl.program_id(0)

    x = x_ref[...]
    logits = jnp.dot(x, w_ref[...], preferred_element_type=jnp.float32)
    logits = logits + b_ref[...]

    m = jnp.max(logits, axis=1, keepdims=True)
    e = jnp.exp(logits - m)
    a = e / jnp.sum(e, axis=1, keepdims=True)

    assign_ref[...] = a

    part = jax.lax.dot_general(
        a, x,
        (((0,), (0,)), ((), ())),
        preferred_element_type=jnp.float32,
    )
    part_s = jax.lax.dot_general(
        a, jnp.ones((BN, 1), jnp.float32),
        (((0,), (0,)), ((), ())),
        preferred_element_type=jnp.float32,
    )

    @pl.when(i == 0)
    def _init():
        pooled_ref[...] = part
        s_ref[...] = part_s

    @pl.when(i > 0)
    def _acc():
        pooled_ref[...] += part
        s_ref[...] += part_s

    @pl.when(i == GRID - 1)
    def _finalize():
        pooled = pooled_ref[...] / s_ref[...]
        pooled_ref[...] = _SELU_SCALE * jnp.where(
            pooled > 0, pooled, _SELU_ALPHA * (jnp.exp(pooled) - 1.0)
        )


def kernel(features, edge_index, W, b):
    del edge_index
    b2 = b.reshape(1, C)
    features_pooled, assignments = pl.pallas_call(
        _dmon_kernel,
        grid=(GRID,),
        in_specs=[
            pl.BlockSpec((BN, D), lambda i: (i, 0)),
            pl.BlockSpec((D, C), lambda i: (0, 0)),
            pl.BlockSpec((1, C), lambda i: (0, 0)),
        ],
        out_specs=[
            pl.BlockSpec((C, D), lambda i: (0, 0)),
            pl.BlockSpec((BN, C), lambda i: (i, 0)),
        ],
        out_shape=[
            jax.ShapeDtypeStruct((C, D), jnp.float32),
            jax.ShapeDtypeStruct((N, C), jnp.float32),
        ],
        scratch_shapes=[pltpu.VMEM((C, 1), jnp.float32)],
        compiler_params=pltpu.CompilerParams(
            dimension_semantics=("arbitrary",),
        ),
    )(features, W, b2)
    return (features_pooled, assignments)
